# baseline (device time: 346729 ns/iter reference)
import jax
import jax.numpy as jnp
from jax import lax
from jax.experimental import pallas as pl
from jax.experimental.pallas import tpu as pltpu

N_DEV = 8
B_PER = 2
SQ = 512
SKV = 512
H_PER = 8
HC = 4
DH = 64
D_MODEL = 768
D_HALF = HC * DH
BLK = 64

_SUCC = (1, 2, 3, 7, 0, 4, 5, 6)
_PRED = (4, 0, 1, 2, 5, 6, 7, 3)


def _perm(table, p):
    out = jnp.int32(table[0])
    for k in range(1, N_DEV):
        out = jnp.where(p == k, jnp.int32(table[k]), out)
    return out


def _attn_group(qf, k_st, v_st, cf_ref, mask, h):
    for b in range(B_PER):
        kc = k_st[b].astype(jnp.bfloat16)
        vc = v_st[b].astype(jnp.bfloat16)
        ctxs = []
        for hl in range(HC):
            q_h = qf[b * SQ:(b + 1) * SQ, hl * DH:(hl + 1) * DH]
            k_h = kc[:, hl * DH:(hl + 1) * DH]
            v_h = vc[:, hl * DH:(hl + 1) * DH]
            s = lax.dot_general(
                q_h, k_h, (((1,), (1,)), ((), ())),
                preferred_element_type=jnp.float32)
            e = jnp.where(mask, jnp.exp(s.astype(jnp.bfloat16)),
                          jnp.bfloat16(0.0))
            denom = jnp.sum(e, axis=-1, dtype=jnp.float32)
            ctx = lax.dot_general(
                e, v_h, (((1,), (0,)), ((), ())),
                preferred_element_type=jnp.float32)
            ctxs.append((ctx / denom[:, None]).astype(jnp.bfloat16))
        cf_ref[pl.ds(b * SQ, SQ), pl.ds(h * D_HALF, D_HALF)] = (
            jnp.concatenate(ctxs, axis=1))


def _body(x_ref, wqa_ref, wqb_ref, woa_ref, wob_ref, k_hbm, v_hbm, out_ref,
          wqa_comm, woa_comm, wqb_comm, wob_comm, cfa_ref, cfb_ref,
          ka_st, va_st, kb_st, vb_st,
          a_wq_send, a_wq_recv, a_wo_send, a_wo_recv,
          b_wq_send, b_wq_recv, b_wo_send, b_wo_recv, st_sems):
    my_pos = lax.axis_index("i")
    left = _perm(_PRED, my_pos)
    right = _perm(_SUCC, my_pos)
    bbase = my_pos * B_PER

    barrier_sem = pltpu.get_barrier_semaphore()
    for nbr in (left, right):
        pl.semaphore_signal(barrier_sem, inc=1, device_id=(nbr,),
                            device_id_type=pl.DeviceIdType.MESH)
    pl.semaphore_wait(barrier_sem, 2)

    def stage(slot, ja, jb):
        cps = []
        for i, (hbm, stg, col) in enumerate((
                (k_hbm, ka_st, ja * (H_PER * DH)),
                (v_hbm, va_st, ja * (H_PER * DH)),
                (k_hbm, kb_st, jb * (H_PER * DH) + D_HALF),
                (v_hbm, vb_st, jb * (H_PER * DH) + D_HALF))):
            cp = pltpu.make_async_copy(
                hbm.at[pl.ds(bbase, B_PER), :, pl.ds(col, D_HALF)],
                stg.at[slot], st_sems.at[i, slot])
            cp.start()
            cps.append(cp)
        return cps

    rows = lax.broadcasted_iota(jnp.int32, (SQ, SKV), 0) // BLK
    cols = lax.broadcasted_iota(jnp.int32, (SQ, SKV), 1) // BLK
    mask = cols <= rows

    x2d = x_ref[...].reshape(B_PER * SQ, D_MODEL)

    woa_comm[0] = woa_ref[...]
    wob_comm[0] = wob_ref[...]

    rdmas = []
    ja = my_pos
    jb = my_pos
    stages = [stage(0, ja, jb)]
    for h in range(N_DEV):
        if h > 0:
            ja = _perm(_PRED, ja)
            jb = _perm(_SUCC, jb)
        if h < N_DEV - 1:
            started = []
            for wq_src0, comm_wq, comm_wo, s_wq, r_wq, s_wo, r_wo, tgt in (
                (wqa_ref, wqa_comm, woa_comm,
                 a_wq_send, a_wq_recv, a_wo_send, a_wo_recv, right),
                (wqb_ref, wqb_comm, wob_comm,
                 b_wq_send, b_wq_recv, b_wo_send, b_wo_recv, left),
            ):
                src_wq = wq_src0 if h == 0 else comm_wq.at[h]
                rdma_wq = pltpu.make_async_remote_copy(
                    src_ref=src_wq, dst_ref=comm_wq.at[h + 1],
                    send_sem=s_wq.at[h], recv_sem=r_wq.at[h + 1],
                    device_id=(tgt,), device_id_type=pl.DeviceIdType.MESH)
                rdma_wo = pltpu.make_async_remote_copy(
                    src_ref=comm_wo.at[h], dst_ref=comm_wo.at[h + 1],
                    send_sem=s_wo.at[h], recv_sem=r_wo.at[h + 1],
                    device_id=(tgt,), device_id_type=pl.DeviceIdType.MESH)
                rdma_wq.start()
                rdma_wo.start()
                started.extend((rdma_wq, rdma_wo))
            rdmas.append(started)
            stages.append(
                stage((h + 1) % 2, _perm(_PRED, ja), _perm(_SUCC, jb)))

        for cp in stages[h]:
            cp.wait()

        wqa_cur = wqa_ref[...] if h == 0 else wqa_comm[h]
        wqb_cur = wqb_ref[...] if h == 0 else wqb_comm[h]
        qfa = jnp.dot(x2d, wqa_cur,
                      preferred_element_type=jnp.float32).astype(jnp.bfloat16)
        qfb = jnp.dot(x2d, wqb_cur,
                      preferred_element_type=jnp.float32).astype(jnp.bfloat16)
        slot = h % 2
        _attn_group(qfa, ka_st.at[slot], va_st.at[slot], cfa_ref, mask, h)
        _attn_group(qfb, kb_st.at[slot], vb_st.at[slot], cfb_ref, mask, h)

        if h < N_DEV - 1:
            for rdma in rdmas[h]:
                rdma.wait_recv()

    for hop_rdmas in rdmas:
        for rdma in hop_rdmas:
            rdma.wait_send()

    woa_all = woa_comm[...].reshape(N_DEV * D_HALF, D_MODEL)
    wob_all = wob_comm[...].reshape(N_DEV * D_HALF, D_MODEL)
    out = jnp.dot(cfa_ref[...], woa_all,
                  preferred_element_type=jnp.float32)
    out = out + jnp.dot(cfb_ref[...], wob_all,
                        preferred_element_type=jnp.float32)
    out_ref[...] = out.reshape(B_PER, SQ, D_MODEL)


def kernel(x, Wq, K_ext, V_ext, Wo):
    k2 = K_ext.reshape(16, SKV, H_PER * N_DEV * DH)
    v2 = V_ext.reshape(16, SKV, H_PER * N_DEV * DH)
    xb = x.astype(jnp.bfloat16)
    wq16 = (Wq * 0.125).astype(jnp.bfloat16)
    wo16 = Wo.astype(jnp.bfloat16)
    wqa, wqb = wq16[:, :D_HALF], wq16[:, D_HALF:]
    woa, wob = wo16[:D_HALF], wo16[D_HALF:]

    return pl.pallas_call(
        _body,
        out_shape=jax.ShapeDtypeStruct((B_PER, SQ, D_MODEL), jnp.float32),
        in_specs=[pl.BlockSpec(memory_space=pltpu.VMEM)] * 5 +
                 [pl.BlockSpec(memory_space=pltpu.MemorySpace.HBM)] * 2,
        out_specs=pl.BlockSpec(memory_space=pltpu.VMEM),
        scratch_shapes=[
            pltpu.VMEM((N_DEV, D_MODEL, D_HALF), jnp.bfloat16),
            pltpu.VMEM((N_DEV, D_HALF, D_MODEL), jnp.bfloat16),
            pltpu.VMEM((N_DEV, D_MODEL, D_HALF), jnp.bfloat16),
            pltpu.VMEM((N_DEV, D_HALF, D_MODEL), jnp.bfloat16),
            pltpu.VMEM((B_PER * SQ, N_DEV * D_HALF), jnp.bfloat16),
            pltpu.VMEM((B_PER * SQ, N_DEV * D_HALF), jnp.bfloat16),
            pltpu.VMEM((2, B_PER, SKV, D_HALF), jnp.float32),
            pltpu.VMEM((2, B_PER, SKV, D_HALF), jnp.float32),
            pltpu.VMEM((2, B_PER, SKV, D_HALF), jnp.float32),
            pltpu.VMEM((2, B_PER, SKV, D_HALF), jnp.float32),
        ] + [pltpu.SemaphoreType.DMA((N_DEV,))] * 8 +
            [pltpu.SemaphoreType.DMA((4, 2))],
        compiler_params=pltpu.CompilerParams(
            collective_id=0, vmem_limit_bytes=100 * 1024 * 1024),
    )(xb, wqa, wqb, woa, wob, k2, v2)
